# baseline (speedup 1.0000x reference)
import functools

import jax
import jax.numpy as jnp
from jax import lax
from jax.experimental import pallas as pl
from jax.experimental.pallas import tpu as pltpu
from jax.experimental.pallas import tpu_sc as plsc

EMB = 16
B_TOTAL = 16384 * 26
NC, NS = 2, 16
NW = NC * NS
B_PER_W = B_TOTAL // NW
CHUNK = 3328
NCHUNK = B_PER_W // CHUNK

_mesh = plsc.VectorSubcoreMesh(core_axis_name="c", subcore_axis_name="s")


@functools.partial(
    pl.kernel,
    mesh=_mesh,
    out_type=jax.ShapeDtypeStruct((B_TOTAL, EMB), jnp.float32),
    scratch_types=[
        pltpu.VMEM((CHUNK,), jnp.int32),
        pltpu.VMEM((CHUNK, EMB), jnp.float32),
        pltpu.SemaphoreType.DMA,
    ],
    compiler_params=pltpu.CompilerParams(use_tc_tiling_on_sc=False),
)
def _gather_kernel(idx_hbm, table_hbm, out_hbm, idx_v, rows_v, sem):
    wid = lax.axis_index("s") * NC + lax.axis_index("c")
    base = wid * B_PER_W
    for c in range(NCHUNK):
        off = base + c * CHUNK
        pltpu.sync_copy(idx_hbm.at[pl.ds(off, CHUNK)], idx_v)
        pltpu.async_copy(table_hbm.at[idx_v], rows_v, sem).wait()
        pltpu.sync_copy(rows_v, out_hbm.at[pl.ds(off, CHUNK)])


def kernel(x, table):
    idx = x.reshape(-1)
    out = _gather_kernel(idx, table)
    return out.reshape(x.shape + (EMB,))


# speedup vs baseline: 1.0029x; 1.0029x over previous
import functools

import jax
import jax.numpy as jnp
from jax import lax
from jax.experimental import pallas as pl
from jax.experimental.pallas import tpu as pltpu
from jax.experimental.pallas import tpu_sc as plsc

EMB = 16
B_TOTAL = 16384 * 26
NC, NS = 2, 16
NW = NC * NS
B_PER_W = B_TOTAL // NW
CHUNK = 3328
NCHUNK = B_PER_W // CHUNK

_mesh = plsc.VectorSubcoreMesh(core_axis_name="c", subcore_axis_name="s")


@functools.partial(
    pl.kernel,
    mesh=_mesh,
    out_type=jax.ShapeDtypeStruct((B_TOTAL, EMB), jnp.float32),
    scratch_types=[
        pltpu.VMEM((2, CHUNK), jnp.int32),
        pltpu.VMEM((2, CHUNK, EMB), jnp.float32),
        pltpu.SemaphoreType.DMA,
        pltpu.SemaphoreType.DMA,
        pltpu.SemaphoreType.DMA,
    ],
    compiler_params=pltpu.CompilerParams(use_tc_tiling_on_sc=False),
)
def _gather_kernel(idx_hbm, table_hbm, out_hbm, idx_v, rows_v, sem_i, sem_g, sem_o):
    wid = lax.axis_index("s") * NC + lax.axis_index("c")
    base = wid * B_PER_W

    def idx_copy(c):
        return pltpu.async_copy(
            idx_hbm.at[pl.ds(base + c * CHUNK, CHUNK)], idx_v.at[c % 2], sem_i)

    def gather(c):
        return pltpu.async_copy(
            table_hbm.at[idx_v.at[c % 2]], rows_v.at[c % 2], sem_g)

    def out_copy(c):
        return pltpu.async_copy(
            rows_v.at[c % 2], out_hbm.at[pl.ds(base + c * CHUNK, CHUNK)], sem_o)

    idx_copy(0).wait()
    g = gather(0)
    if NCHUNK > 1:
        i_next = idx_copy(1)
    outs = []
    for c in range(NCHUNK):
        g.wait()
        outs.append(out_copy(c))
        if c >= 1:
            outs[c - 1].wait()
        if c + 1 < NCHUNK:
            i_next.wait()
            g = gather(c + 1)
        if c + 2 < NCHUNK:
            i_next = idx_copy(c + 2)
    outs[NCHUNK - 1].wait()


def kernel(x, table):
    idx = x.reshape(-1)
    out = _gather_kernel(idx, table)
    return out.reshape(x.shape + (EMB,))


# speedup vs baseline: 1.5915x; 1.5868x over previous
import functools

import jax
import jax.numpy as jnp
from jax import lax
from jax.experimental import pallas as pl
from jax.experimental.pallas import tpu as pltpu
from jax.experimental.pallas import tpu_sc as plsc

EMB = 16
FIELDS = 26
BATCH = 16384
B_TOTAL = BATCH * FIELDS
NC, NS = 2, 16
NW = NC * NS
NBH = BATCH // (128 * NW)
CHUNK = 128 * FIELDS

_mesh = plsc.VectorSubcoreMesh(core_axis_name="c", subcore_axis_name="s")


@functools.partial(
    pl.kernel,
    mesh=_mesh,
    out_type=jax.ShapeDtypeStruct((FIELDS, 2, BATCH // 128, 8 * 128), jnp.float32),
    scratch_types=[
        pltpu.VMEM((2, CHUNK), jnp.int32),
        pltpu.VMEM((2, CHUNK, EMB), jnp.float32),
        pltpu.VMEM((2 * EMB * 128,), jnp.float32),
        pltpu.SemaphoreType.DMA,
        pltpu.SemaphoreType.DMA,
        pltpu.SemaphoreType.DMA,
    ],
    compiler_params=pltpu.CompilerParams(
        use_tc_tiling_on_sc=False, needs_layout_passes=False),
)
def _emb_kernel(idx_hbm, table_hbm, out_hbm, idx_v, rows_v, tr_v, sem_i, sem_g, sem_o):
    wid = lax.axis_index("s") * NC + lax.axis_index("c")
    lane128 = lax.iota(jnp.int32, 16) * 128

    def idx_copy(c):
        bh = wid * NBH + c
        return pltpu.async_copy(
            idx_hbm.at[pl.ds(bh * CHUNK, CHUNK)], idx_v.at[c % 2], sem_i)

    def gather(c):
        return pltpu.async_copy(
            table_hbm.at[idx_v.at[c % 2]], rows_v.at[c % 2], sem_g)

    def transpose_and_store(c):
        bh = wid * NBH + c
        rbuf = c % 2
        pending = []
        for f in range(FIELDS):
            tbuf = f % 2
            def body(bl, _):
                vreg = rows_v[rbuf, bl * FIELDS + f, :]
                plsc.store_scatter(tr_v, [lane128 + (tbuf * 2048 + bl)], vreg)
                return ()
            lax.fori_loop(0, 128, body, (), unroll=8)
            if f >= 2:
                for cp in pending.pop(0):
                    cp.wait()
            cps = []
            for dh in range(2):
                cps.append(pltpu.async_copy(
                    tr_v.at[pl.ds(tbuf * 2048 + dh * 1024, 1024)],
                    out_hbm.at[f, dh, bh],
                    sem_o))
            pending.append(cps)
        for cps in pending:
            for cp in cps:
                cp.wait()

    idx_copy(0).wait()
    g = gather(0)
    i_next = idx_copy(1)
    for c in range(NBH):
        g.wait()
        if c + 1 < NBH:
            i_next.wait()
            g = gather(c + 1)
        if c + 2 < NBH:
            i_next = idx_copy(c + 2)
        transpose_and_store(c)


def kernel(x, table):
    idx = x.reshape(-1)
    out4 = _emb_kernel(idx, table)
    out = (
        out4.reshape(FIELDS, 2, BATCH // 128, 8, 128)
        .transpose(2, 4, 0, 1, 3)
        .reshape(BATCH, FIELDS, EMB)
    )
    return out
